# Initial kernel scaffold; baseline (speedup 1.0000x reference)
import jax
import jax.numpy as jnp
from jax.experimental import pallas as pl


def kernel(x, emb_weight):
    raise NotImplementedError("write your pallas kernel here")


# speedup vs baseline: 1.3848x; 1.3848x over previous
import functools

import jax
import jax.numpy as jnp
from jax import lax
from jax.experimental import pallas as pl
from jax.experimental.pallas import tpu as pltpu
from jax.experimental.pallas import tpu_sc as plsc

NC = 2
NS = 16
NW = NC * NS
CHUNK = 16


def _build_normalize(vocab, dim):
    blk = 256

    def norm_kernel(w_ref, o_ref):
        v = w_ref[...]
        ss = jnp.sum(v * v, axis=1, keepdims=True)
        o_ref[...] = v / jnp.sqrt(jnp.maximum(ss, 1e-24))

    return pl.pallas_call(
        norm_kernel,
        grid=(vocab // blk,),
        in_specs=[pl.BlockSpec((blk, dim), lambda i: (i, 0))],
        out_specs=pl.BlockSpec((blk, dim), lambda i: (i, 0)),
        out_shape=jax.ShapeDtypeStruct((vocab, dim), jnp.float32),
    )


def _build_gather(rows, dim):
    per_w = rows // NW
    nchunk = per_w // CHUNK
    mesh = plsc.VectorSubcoreMesh(core_axis_name="c", subcore_axis_name="s")

    @functools.partial(
        pl.kernel,
        mesh=mesh,
        out_type=jax.ShapeDtypeStruct((rows, dim), jnp.float32),
        scratch_types=[
            pltpu.VMEM((nchunk, CHUNK), jnp.int32),
            pltpu.VMEM((CHUNK, dim), jnp.float32),
            pltpu.SemaphoreType.DMA,
        ],
    )
    def gather_kernel(x_hbm, table_hbm, out_hbm, idx_v, buf, gsem):
        cid = lax.axis_index("c")
        sid = lax.axis_index("s")
        wid = sid * NC + cid
        base = wid * per_w

        pltpu.sync_copy(x_hbm.at[wid], idx_v)

        def chunk_body(g, carry):
            pltpu.async_copy(table_hbm.at[idx_v.at[g]], buf, gsem).wait()
            pltpu.sync_copy(buf, out_hbm.at[pl.ds(base + g * CHUNK, CHUNK)])
            return carry

        lax.fori_loop(0, nchunk, chunk_body, 0)

    return gather_kernel


_CACHE = {}


def kernel(x, emb_weight):
    b, s = x.shape
    vocab, dim = emb_weight.shape
    rows = b * s
    key = (rows, vocab, dim)
    if key not in _CACHE:
        _CACHE[key] = (_build_normalize(vocab, dim), _build_gather(rows, dim))
    normalize, gather = _CACHE[key]
    table_n = normalize(emb_weight)
    xw = x.reshape(NW, rows // (NW * CHUNK), CHUNK).astype(jnp.int32)
    out = gather(xw, table_n)
    return out.reshape(b, s, dim)


# speedup vs baseline: 1.5898x; 1.1480x over previous
import functools

import jax
import jax.numpy as jnp
from jax import lax
from jax.experimental import pallas as pl
from jax.experimental.pallas import tpu as pltpu
from jax.experimental.pallas import tpu_sc as plsc

NC = 2
NS = 16
NW = NC * NS
CHUNK = 16


def _build_normalize(vocab, dim):
    blk = 256

    def norm_kernel(w_ref, o_ref):
        v = w_ref[...]
        ss = jnp.sum(v * v, axis=1, keepdims=True)
        o_ref[...] = v / jnp.sqrt(jnp.maximum(ss, 1e-24))

    return pl.pallas_call(
        norm_kernel,
        grid=(vocab // blk,),
        in_specs=[pl.BlockSpec((blk, dim), lambda i: (i, 0))],
        out_specs=pl.BlockSpec((blk, dim), lambda i: (i, 0)),
        out_shape=jax.ShapeDtypeStruct((vocab, dim), jnp.float32),
    )


def _build_gather(rows, dim):
    per_w = rows // NW
    nchunk = per_w // CHUNK
    mesh = plsc.VectorSubcoreMesh(core_axis_name="c", subcore_axis_name="s")

    @functools.partial(
        pl.kernel,
        mesh=mesh,
        out_type=jax.ShapeDtypeStruct((rows, dim), jnp.float32),
        scratch_types=[
            pltpu.VMEM((nchunk, CHUNK), jnp.int32),
            pltpu.VMEM((CHUNK, dim), jnp.float32),
            pltpu.VMEM((CHUNK, dim), jnp.float32),
            pltpu.SemaphoreType.DMA,
            pltpu.SemaphoreType.DMA,
            pltpu.SemaphoreType.DMA,
            pltpu.SemaphoreType.DMA,
        ],
    )
    def gather_kernel(x_hbm, table_hbm, out_hbm, idx_v, buf0, buf1,
                      gsem0, gsem1, wsem0, wsem1):
        cid = lax.axis_index("c")
        sid = lax.axis_index("s")
        wid = sid * NC + cid
        base = wid * per_w

        bufs = (buf0, buf1)
        gsems = (gsem0, gsem1)
        wsems = (wsem0, wsem1)

        pltpu.sync_copy(x_hbm.at[wid], idx_v)

        def start_gather(g, b):
            pltpu.async_copy(table_hbm.at[idx_v.at[g]], bufs[b], gsems[b])

        def wait_gather(b):
            pltpu.make_async_copy(
                out_hbm.at[pl.ds(base, CHUNK)], bufs[b], gsems[b]
            ).wait()

        def start_scatter(g, b):
            pltpu.async_copy(
                bufs[b], out_hbm.at[pl.ds(base + g * CHUNK, CHUNK)], wsems[b]
            )

        def wait_scatter(b):
            pltpu.make_async_copy(
                bufs[b], out_hbm.at[pl.ds(base, CHUNK)], wsems[b]
            ).wait()

        def phase(g, b):
            wait_gather(b)
            wait_scatter(1 - b)
            start_gather(g + 1, 1 - b)
            start_scatter(g, b)

        start_gather(0, 0)
        wait_gather(0)
        start_gather(1, 1)
        start_scatter(0, 0)

        def core(i, carry):
            phase(1 + 2 * i, 1)
            phase(2 + 2 * i, 0)
            return carry

        lax.fori_loop(0, (nchunk - 2) // 2, core, 0)

        wait_gather(1)
        wait_scatter(0)
        start_scatter(nchunk - 1, 1)
        wait_scatter(1)

    return gather_kernel


_CACHE = {}


def kernel(x, emb_weight):
    b, s = x.shape
    vocab, dim = emb_weight.shape
    rows = b * s
    key = (rows, vocab, dim)
    if key not in _CACHE:
        _CACHE[key] = (_build_normalize(vocab, dim), _build_gather(rows, dim))
    normalize, gather = _CACHE[key]
    table_n = normalize(emb_weight)
    xw = x.reshape(NW, rows // (NW * CHUNK), CHUNK).astype(jnp.int32)
    out = gather(xw, table_n)
    return out.reshape(b, s, dim)


# speedup vs baseline: 1.6168x; 1.0170x over previous
import functools

import jax
import jax.numpy as jnp
from jax import lax
from jax.experimental import pallas as pl
from jax.experimental.pallas import tpu as pltpu
from jax.experimental.pallas import tpu_sc as plsc

NC = 2
NS = 16
NW = NC * NS
CHUNK = 16


def _build_normalize(vocab, dim):
    blk = 256

    def norm_kernel(w_ref, o_ref):
        v = w_ref[...]
        ss = jnp.sum(v * v, axis=1, keepdims=True)
        o_ref[...] = v / jnp.sqrt(jnp.maximum(ss, 1e-24))

    return pl.pallas_call(
        norm_kernel,
        grid=(vocab // blk,),
        in_specs=[pl.BlockSpec((blk, dim), lambda i: (i, 0))],
        out_specs=pl.BlockSpec((blk, dim), lambda i: (i, 0)),
        out_shape=jax.ShapeDtypeStruct((vocab, dim), jnp.float32),
    )


def _build_gather(rows, dim):
    per_w = rows // NW
    nchunk = per_w // CHUNK
    mesh = plsc.VectorSubcoreMesh(core_axis_name="c", subcore_axis_name="s")

    @functools.partial(
        pl.kernel,
        mesh=mesh,
        out_type=jax.ShapeDtypeStruct((rows, dim), jnp.float32),
        scratch_types=[
            pltpu.VMEM((nchunk, CHUNK), jnp.int32),
            pltpu.VMEM((CHUNK, dim), jnp.float32),
            pltpu.VMEM((CHUNK, dim), jnp.float32),
            pltpu.VMEM((CHUNK, dim), jnp.float32),
            pltpu.SemaphoreType.DMA,
            pltpu.SemaphoreType.DMA,
            pltpu.SemaphoreType.DMA,
            pltpu.SemaphoreType.DMA,
            pltpu.SemaphoreType.DMA,
            pltpu.SemaphoreType.DMA,
        ],
    )
    def gather_kernel(x_hbm, table_hbm, out_hbm, idx_v, buf0, buf1, buf2,
                      gsem0, gsem1, gsem2, wsem0, wsem1, wsem2):
        cid = lax.axis_index("c")
        sid = lax.axis_index("s")
        wid = sid * NC + cid
        base = wid * per_w
        n = nchunk

        bufs = (buf0, buf1, buf2)
        gsems = (gsem0, gsem1, gsem2)
        wsems = (wsem0, wsem1, wsem2)

        pltpu.sync_copy(x_hbm.at[wid], idx_v)

        def start_gather(g, b):
            pltpu.async_copy(table_hbm.at[idx_v.at[g]], bufs[b], gsems[b])

        def wait_gather(b):
            pltpu.make_async_copy(
                out_hbm.at[pl.ds(base, CHUNK)], bufs[b], gsems[b]
            ).wait()

        def start_scatter(g, b):
            pltpu.async_copy(
                bufs[b], out_hbm.at[pl.ds(base + g * CHUNK, CHUNK)], wsems[b]
            )

        def wait_scatter(b):
            pltpu.make_async_copy(
                bufs[b], out_hbm.at[pl.ds(base, CHUNK)], wsems[b]
            ).wait()

        def phase(g, b, first=False, last=False):
            wait_gather(b)
            start_scatter(g, b)
            if not first:
                wait_scatter((b + 2) % 3)
            if not last:
                start_gather(g + 2, (b + 2) % 3)

        start_gather(0, 0)
        start_gather(1, 1)
        phase(0, 0, first=True)
        phase(1, 1)
        phase(2, 2)

        def core(i, carry):
            g0 = 3 * i

            def guarded(g, b):
                wait_gather(b)
                start_scatter(g, b)
                wait_scatter((b + 2) % 3)

                @pl.when(g + 2 < n)
                def _():
                    start_gather(g + 2, (b + 2) % 3)

            guarded(g0, 0)
            guarded(g0 + 1, 1)
            guarded(g0 + 2, 2)
            return carry

        lax.fori_loop(1, (n - 1) // 3, core, 0)

        wait_gather(0)
        start_scatter(n - 1, 0)
        wait_scatter(2)
        wait_scatter(0)

    return gather_kernel


_CACHE = {}


def kernel(x, emb_weight):
    b, s = x.shape
    vocab, dim = emb_weight.shape
    rows = b * s
    key = (rows, vocab, dim)
    if key not in _CACHE:
        _CACHE[key] = (_build_normalize(vocab, dim), _build_gather(rows, dim))
    normalize, gather = _CACHE[key]
    table_n = normalize(emb_weight)
    xw = x.reshape(NW, rows // (NW * CHUNK), CHUNK).astype(jnp.int32)
    out = gather(xw, table_n)
    return out.reshape(b, s, dim)
